# baseline (speedup 1.0000x reference)
import functools

import jax
import jax.numpy as jnp
from jax import lax
from jax.experimental import pallas as pl
from jax.experimental.pallas import tpu as pltpu
from jax.experimental.pallas import tpu_sc as plsc

N = 10000
E = 320000
D_FEAT = 128
D_EDGE = 16
OUT = 8
ROW = 144

N_TILES = 32
TILES_PER_REL = 16
EDGES_PER_TILE = E // TILES_PER_REL
BLK = 80
N_BLOCKS = EDGES_PER_TILE // BLK


def _table_body(x_ref, w_ref, o_ref):
    o_ref[0] = jnp.dot(x_ref[0], w_ref[0], preferred_element_type=jnp.float32)


def _build_tables(xs, wcat):
    nb = 25
    bn = N // nb
    return pl.pallas_call(
        _table_body,
        grid=(4, nb),
        in_specs=[
            pl.BlockSpec((1, bn, D_FEAT), lambda r, b: (r % 2, b, 0)),
            pl.BlockSpec((1, D_FEAT, ROW), lambda r, b: (r, 0, 0)),
        ],
        out_specs=pl.BlockSpec((1, bn, ROW), lambda r, b: (r, b, 0)),
        out_shape=jax.ShapeDtypeStruct((4, N, ROW), jnp.float32),
    )(xs, wcat)


def _edge_pass(table_ref, ei_ref, ea_ref, acc, bufs, sub, rbase):
    (src_v, dst_v, ea_v, rows_v, s_src, s_dst, s_ea, s_row) = bufs
    iota = lax.iota(jnp.int32, 16)
    mask_lo = iota < 8
    mask_hi = jnp.logical_not(mask_lo)
    c_lo = jnp.where(mask_lo, iota, 0)
    c_hi = jnp.where(mask_lo, 0, iota - 8)
    cexp = [jnp.where(mask_lo, 2 * j, 2 * j + 1) for j in range(8)]

    def issue_src(b, sl):
        base = sub * EDGES_PER_TILE + b * BLK
        pltpu.async_copy(ei_ref.at[pl.ds(base, BLK)], src_v[sl], s_src[sl])

    def issue_de(b, sl):
        base = sub * EDGES_PER_TILE + b * BLK
        pltpu.async_copy(ei_ref.at[pl.ds(E + base, BLK)], dst_v[sl],
                         s_dst[sl])
        pltpu.async_copy(ea_ref.at[pl.ds(base * D_EDGE, BLK * D_EDGE)],
                         ea_v[sl], s_ea[sl])

    def start_gather(sl):
        pltpu.make_async_copy(ei_ref.at[pl.ds(0, BLK)], src_v[sl],
                              s_src[sl]).wait()
        for t in range(BLK // 16):
            s2 = pl.ds(t * 16, 16)
            src_v[sl][s2] = src_v[sl][s2] + rbase
        pltpu.async_copy(table_ref.at[src_v[sl]], rows_v[sl], s_row[sl])

    def process(sl):
        pltpu.make_async_copy(table_ref.at[src_v[sl]], rows_v[sl],
                              s_row[sl]).wait()
        pltpu.make_async_copy(ei_ref.at[pl.ds(0, BLK)], dst_v[sl],
                              s_dst[sl]).wait()
        pltpu.make_async_copy(ei_ref.at[pl.ds(0, BLK * D_EDGE)], ea_v[sl],
                              s_ea[sl]).wait()

        def group_body(g, carry2):
            dv8 = dst_v[sl][pl.ds(g * 16, 16)] * 8
            for u in range(16):
                e = g * 16 + u
                d8v = jnp.broadcast_to(dv8[u], (16,))
                e16 = jnp.broadcast_to(e * 16, (16,))
                p = [plsc.load_gather(ea_v[sl], [e16 + cexp[j]])
                     * rows_v[sl][e, pl.ds(16 * j, 16)] for j in range(8)]
                p.append(rows_v[sl][e, pl.ds(128, 16)])
                s = (((p[0] + p[1]) + (p[2] + p[3]))
                     + ((p[4] + p[5]) + (p[6] + p[7])) + p[8])
                plsc.addupdate_scatter(acc, [d8v + c_lo], s, mask=mask_lo)
                plsc.addupdate_scatter(acc, [d8v + c_hi], s, mask=mask_hi)
            return carry2

        lax.fori_loop(0, BLK // 16, group_body, 0)

    issue_src(0, 0)
    issue_de(0, 0)
    issue_src(1, 1)
    issue_de(1, 1)
    start_gather(0)

    def body2(i, carry):
        for k in range(2):
            b = 2 * i + k

            @pl.when(b + 2 < N_BLOCKS)
            def _():
                issue_src(b + 2, k)

            @pl.when(b + 1 < N_BLOCKS)
            def _():
                start_gather(1 - k)

            process(k)

            @pl.when(b + 2 < N_BLOCKS)
            def _():
                issue_de(b + 2, k)
        return carry

    lax.fori_loop(0, N_BLOCKS // 2, body2, 0)


def _zero_acc(acc):
    zeros16 = jnp.zeros((16,), jnp.float32)

    def zero_body(i, carry):
        acc[pl.ds(i * 16, 16)] = zeros16
        return carry

    lax.fori_loop(0, (N * OUT) // 16, zero_body, 0)


def _sc_body(table_ref, ei0, ei1, ei2, ei3, ea0, ea1, ea2, ea3, part_ref,
             acc, src0, src1, dst0, dst1, ea0v, ea1v, rows0, rows1,
             ss0, ss1, sd0, sd1, se0, se1, sr0, sr1):
    c = lax.axis_index("c")
    s = lax.axis_index("s")
    bufs = ((src0, src1), (dst0, dst1), (ea0v, ea1v), (rows0, rows1),
            (ss0, ss1), (sd0, sd1), (se0, se1), (sr0, sr1))

    eis = [ei0, ei1, ei2, ei3]
    eas = [ea0, ea1, ea2, ea3]
    for ci in range(2):
        @pl.when(c == ci)
        def _(ci=ci):
            for phase in range(2):
                r = ci * 2 + phase
                _zero_acc(acc)
                _edge_pass(table_ref, eis[r], eas[r], acc, bufs, s, r * N)
                pltpu.sync_copy(acc, part_ref.at[r, s])


def _sc_edges(table, ei_list, ea_list):
    mesh = plsc.VectorSubcoreMesh(core_axis_name="c", subcore_axis_name="s")
    fn = pl.kernel(
        _sc_body,
        out_type=jax.ShapeDtypeStruct((4, TILES_PER_REL, N * OUT),
                                      jnp.float32),
        mesh=mesh,
        compiler_params=pltpu.CompilerParams(needs_layout_passes=False,
                                             use_tc_tiling_on_sc=False),
        scratch_types=[
            pltpu.VMEM((N * OUT,), jnp.float32),
            pltpu.VMEM((BLK,), jnp.int32),
            pltpu.VMEM((BLK,), jnp.int32),
            pltpu.VMEM((BLK,), jnp.int32),
            pltpu.VMEM((BLK,), jnp.int32),
            pltpu.VMEM((BLK * D_EDGE,), jnp.float32),
            pltpu.VMEM((BLK * D_EDGE,), jnp.float32),
            pltpu.VMEM((BLK, ROW), jnp.float32),
            pltpu.VMEM((BLK, ROW), jnp.float32),
            pltpu.SemaphoreType.DMA,
            pltpu.SemaphoreType.DMA,
            pltpu.SemaphoreType.DMA,
            pltpu.SemaphoreType.DMA,
            pltpu.SemaphoreType.DMA,
            pltpu.SemaphoreType.DMA,
            pltpu.SemaphoreType.DMA,
            pltpu.SemaphoreType.DMA,
        ],
    )
    return fn(table, *ei_list, *ea_list)


def _combine_body(x_ref, wr_ref, b_ref, p_ref, o_ref):
    d = jnp.dot(x_ref[...], wr_ref[...], preferred_element_type=jnp.float32)
    cols = []
    for r in range(4):
        cols.append(jnp.sum(p_ref[r], axis=0))
    o_ref[...] = d + b_ref[0:1, :] + jnp.concatenate(cols, axis=1)


def _combine(x_ext, wroot, biasp, part):
    nb = 25
    bn = N // nb
    return pl.pallas_call(
        _combine_body,
        grid=(nb,),
        in_specs=[
            pl.BlockSpec((bn, D_FEAT), lambda b: (b, 0)),
            pl.BlockSpec((D_FEAT, 4 * OUT), lambda b: (0, 0)),
            pl.BlockSpec((8, 4 * OUT), lambda b: (0, 0)),
            pl.BlockSpec((4, TILES_PER_REL, bn, OUT), lambda b: (0, 0, b, 0)),
        ],
        out_specs=pl.BlockSpec((bn, 4 * OUT), lambda b: (b, 0)),
        out_shape=jax.ShapeDtypeStruct((N, 4 * OUT), jnp.float32),
    )(x_ext, wroot, biasp, part)


def kernel(x_ind, x_org, x_ext,
           ei_ind_txn_ext, ei_org_txn_ext, ei_ind_rev_txn_ext,
           ei_org_rev_txn_ext,
           ea_ind_txn_ext, ea_org_txn_ext, ea_ind_rev_txn_ext,
           ea_org_rev_txn_ext,
           W_nn_1, b_nn_1, W_root_1, bias_1,
           W_nn_2, b_nn_2, W_root_2, bias_2,
           W_nn_3, b_nn_3, W_root_3, bias_3,
           W_nn_4, b_nn_4, W_root_4, bias_4):
    wnns = [W_nn_1, W_nn_2, W_nn_3, W_nn_4]
    bnns = [b_nn_1, b_nn_2, b_nn_3, b_nn_4]

    wcats = []
    for wnn, bnn in zip(wnns, bnns):
        wt = wnn.reshape(D_EDGE, D_FEAT, OUT).transpose(1, 0, 2)
        wt = wt.reshape(D_FEAT, D_EDGE * OUT)
        bt = bnn.reshape(D_FEAT, OUT)
        wcats.append(jnp.concatenate(
            [wt, bt, jnp.zeros((D_FEAT, OUT), jnp.float32)], axis=1))
    wcat = jnp.stack(wcats)
    xs = jnp.stack([x_ind, x_org])
    wroot = jnp.concatenate([W_root_1, W_root_2, W_root_3, W_root_4], axis=1)
    bias = jnp.concatenate([bias_1, bias_2, bias_3, bias_4])
    biasp = jnp.broadcast_to(bias[None, :], (8, 4 * OUT))

    table = _build_tables(xs, wcat)
    part = _sc_edges(
        table.reshape(4 * N, ROW),
        [ei_ind_txn_ext.reshape(2 * E), ei_org_txn_ext.reshape(2 * E),
         ei_ind_rev_txn_ext.reshape(2 * E), ei_org_rev_txn_ext.reshape(2 * E)],
        [ea_ind_txn_ext.reshape(E * D_EDGE), ea_org_txn_ext.reshape(E * D_EDGE),
         ea_ind_rev_txn_ext.reshape(E * D_EDGE),
         ea_org_rev_txn_ext.reshape(E * D_EDGE)])
    part = part.reshape(4, TILES_PER_REL, N, OUT)
    return _combine(x_ext, wroot, biasp, part)


# speedup vs baseline: 1.3780x; 1.3780x over previous
import functools

import jax
import jax.numpy as jnp
from jax import lax
from jax.experimental import pallas as pl
from jax.experimental.pallas import tpu as pltpu
from jax.experimental.pallas import tpu_sc as plsc

N = 10000
E = 320000
D_FEAT = 128
D_EDGE = 16
OUT = 8
ROW = 144

N_TILES = 32
TILES_PER_REL = 16
EDGES_PER_TILE = E // TILES_PER_REL
BLK = 80
N_BLOCKS = EDGES_PER_TILE // BLK


def _table_body(x_ref, w_ref, o_ref):
    o_ref[0] = jnp.dot(x_ref[0], w_ref[0], preferred_element_type=jnp.float32)


def _build_tables(xs, wcat):
    nb = 25
    bn = N // nb
    return pl.pallas_call(
        _table_body,
        grid=(4, nb),
        in_specs=[
            pl.BlockSpec((1, bn, D_FEAT), lambda r, b: (r % 2, b, 0)),
            pl.BlockSpec((1, D_FEAT, ROW), lambda r, b: (r, 0, 0)),
        ],
        out_specs=pl.BlockSpec((1, bn, ROW), lambda r, b: (r, b, 0)),
        out_shape=jax.ShapeDtypeStruct((4, N, ROW), jnp.float32),
    )(xs, wcat)


def _edge_pass(table_ref, ei_ref, ea_ref, part_ref, bufs, sub, r, zbuf,
               shared_tab, shared_acc):
    (src_v, dst_v, didx_v, ea_v, rows_v, res_v,
     s_src, s_dst, s_ea, s_row, s_res) = bufs
    chunk = N // 16
    pltpu.sync_copy(table_ref.at[pl.ds(r * N + sub * chunk, chunk)],
                    shared_tab.at[pl.ds(sub * chunk, chunk)])
    pltpu.sync_copy(zbuf.at[pl.ds(0, chunk)],
                    shared_acc.at[pl.ds(sub * chunk, chunk)])
    plsc.subcore_barrier()

    iota = lax.iota(jnp.int32, 16)
    mask_lo = iota < 8
    rot8 = jnp.where(mask_lo, iota + 8, iota - 8)
    cexp = [jnp.where(mask_lo, 2 * j, 2 * j + 1) for j in range(8)]

    def issue_src(b, sl):
        base = sub * EDGES_PER_TILE + b * BLK
        pltpu.async_copy(ei_ref.at[pl.ds(base, BLK)], src_v[sl], s_src[sl])

    def issue_de(b, sl):
        base = sub * EDGES_PER_TILE + b * BLK
        pltpu.async_copy(ei_ref.at[pl.ds(E + base, BLK)], dst_v[sl],
                         s_dst[sl])
        pltpu.async_copy(ea_ref.at[pl.ds(base * D_EDGE, BLK * D_EDGE)],
                         ea_v[sl], s_ea[sl])

    def start_gather(sl):
        pltpu.make_async_copy(ei_ref.at[pl.ds(0, BLK)], src_v[sl],
                              s_src[sl]).wait()
        pltpu.async_copy(shared_tab.at[src_v[sl]], rows_v[sl], s_row[sl])

    def wait_scatter(sl):
        pltpu.make_async_copy(res_v[sl], shared_acc.at[didx_v[sl]],
                              s_res[sl]).wait()

    def process(sl, do_wait):
        pltpu.make_async_copy(shared_tab.at[src_v[sl]], rows_v[sl],
                              s_row[sl]).wait()
        pltpu.make_async_copy(ei_ref.at[pl.ds(0, BLK)], dst_v[sl],
                              s_dst[sl]).wait()
        pltpu.make_async_copy(ei_ref.at[pl.ds(0, BLK * D_EDGE)], ea_v[sl],
                              s_ea[sl]).wait()

        @pl.when(do_wait)
        def _():
            wait_scatter(sl)

        def group_body(g, carry2):
            sl16 = pl.ds(g * 16, 16)
            didx_v[sl][sl16] = dst_v[sl][sl16]
            for u in range(16):
                e = g * 16 + u
                erow = jnp.broadcast_to(e, (16,))
                ea_row = ea_v[sl][pl.ds(e * 16, 16)]
                p = [jnp.take_along_axis(ea_row, cexp[j], axis=0)
                     * rows_v[sl][e, pl.ds(16 * j, 16)] for j in range(8)]
                p.append(rows_v[sl][e, pl.ds(128, 16)])
                s = (((p[0] + p[1]) + (p[2] + p[3]))
                     + ((p[4] + p[5]) + (p[6] + p[7])) + p[8])
                f = s + jnp.take_along_axis(s, rot8, axis=0)
                plsc.store_scatter(res_v[sl], [erow, iota], f, mask=mask_lo)
            return carry2

        lax.fori_loop(0, BLK // 16, group_body, 0)
        pltpu.async_copy(res_v[sl], shared_acc.at[didx_v[sl]], s_res[sl],
                         add=True)

    issue_src(0, 0)
    issue_de(0, 0)
    issue_src(1, 1)
    issue_de(1, 1)
    start_gather(0)

    def body2(i, carry):
        for k in range(2):
            b = 2 * i + k

            @pl.when(b + 2 < N_BLOCKS)
            def _():
                issue_src(b + 2, k)

            @pl.when(b + 1 < N_BLOCKS)
            def _():
                start_gather(1 - k)

            process(k, b >= 2)

            @pl.when(b + 2 < N_BLOCKS)
            def _():
                issue_de(b + 2, k)
        return carry

    lax.fori_loop(0, N_BLOCKS // 2, body2, 0)
    wait_scatter(0)
    wait_scatter(1)
    plsc.subcore_barrier()
    pltpu.sync_copy(shared_acc.at[pl.ds(sub * chunk, chunk)],
                    part_ref.at[r, pl.ds(sub * chunk, chunk)])


def _sc_body(table_ref, ei0, ei1, ei2, ei3, ea0, ea1, ea2, ea3, part_ref,
             src0, src1, dst0, dst1, di0, di1, ea0v, ea1v, rows0, rows1,
             res0, res1, zbuf, shared_tab, shared_acc,
             ss0, ss1, sd0, sd1, se0, se1, sr0, sr1, sx0, sx1):
    c = lax.axis_index("c")
    s = lax.axis_index("s")
    bufs = ((src0, src1), (dst0, dst1), (di0, di1), (ea0v, ea1v),
            (rows0, rows1), (res0, res1),
            (ss0, ss1), (sd0, sd1), (se0, se1), (sr0, sr1), (sx0, sx1))

    zeros16 = jnp.zeros((16,), jnp.float32)
    iota = lax.iota(jnp.int32, 16)
    lo = iota < 8
    zcol = jnp.where(lo, iota, iota - 8)

    def zero_body(i, carry):
        zrow = jnp.where(lo, 2 * i, 2 * i + 1)
        plsc.store_scatter(zbuf, [zrow, zcol], zeros16)
        return carry

    lax.fori_loop(0, 320, zero_body, 0)

    eis = [ei0, ei1, ei2, ei3]
    eas = [ea0, ea1, ea2, ea3]
    for ci in range(2):
        @pl.when(c == ci)
        def _(ci=ci):
            for phase in range(2):
                r = ci * 2 + phase
                _edge_pass(table_ref, eis[r], eas[r], part_ref, bufs, s, r,
                           zbuf, shared_tab, shared_acc)


def _sc_edges(table, ei_list, ea_list):
    mesh = plsc.VectorSubcoreMesh(core_axis_name="c", subcore_axis_name="s")
    fn = pl.kernel(
        _sc_body,
        out_type=jax.ShapeDtypeStruct((4, N, OUT), jnp.float32),
        mesh=mesh,
        compiler_params=pltpu.CompilerParams(needs_layout_passes=False,
                                             use_tc_tiling_on_sc=False),
        scratch_types=[
            pltpu.VMEM((BLK,), jnp.int32),
            pltpu.VMEM((BLK,), jnp.int32),
            pltpu.VMEM((BLK,), jnp.int32),
            pltpu.VMEM((BLK,), jnp.int32),
            pltpu.VMEM((BLK,), jnp.int32),
            pltpu.VMEM((BLK,), jnp.int32),
            pltpu.VMEM((BLK * D_EDGE,), jnp.float32),
            pltpu.VMEM((BLK * D_EDGE,), jnp.float32),
            pltpu.VMEM((BLK, ROW), jnp.float32),
            pltpu.VMEM((BLK, ROW), jnp.float32),
            pltpu.VMEM((BLK, OUT), jnp.float32),
            pltpu.VMEM((BLK, OUT), jnp.float32),
            pltpu.VMEM((640, OUT), jnp.float32),
            pltpu.VMEM_SHARED((N, ROW), jnp.float32),
            pltpu.VMEM_SHARED((N, OUT), jnp.float32),
            pltpu.SemaphoreType.DMA,
            pltpu.SemaphoreType.DMA,
            pltpu.SemaphoreType.DMA,
            pltpu.SemaphoreType.DMA,
            pltpu.SemaphoreType.DMA,
            pltpu.SemaphoreType.DMA,
            pltpu.SemaphoreType.DMA,
            pltpu.SemaphoreType.DMA,
            pltpu.SemaphoreType.DMA,
            pltpu.SemaphoreType.DMA,
        ],
    )
    return fn(table, *ei_list, *ea_list)


def _combine_body(x_ref, wr_ref, b_ref, p_ref, o_ref):
    d = jnp.dot(x_ref[...], wr_ref[...], preferred_element_type=jnp.float32)
    cols = [p_ref[r] for r in range(4)]
    o_ref[...] = d + b_ref[0:1, :] + jnp.concatenate(cols, axis=1)


def _combine(x_ext, wroot, biasp, part):
    nb = 25
    bn = N // nb
    return pl.pallas_call(
        _combine_body,
        grid=(nb,),
        in_specs=[
            pl.BlockSpec((bn, D_FEAT), lambda b: (b, 0)),
            pl.BlockSpec((D_FEAT, 4 * OUT), lambda b: (0, 0)),
            pl.BlockSpec((8, 4 * OUT), lambda b: (0, 0)),
            pl.BlockSpec((4, bn, OUT), lambda b: (0, b, 0)),
        ],
        out_specs=pl.BlockSpec((bn, 4 * OUT), lambda b: (b, 0)),
        out_shape=jax.ShapeDtypeStruct((N, 4 * OUT), jnp.float32),
    )(x_ext, wroot, biasp, part)


def kernel(x_ind, x_org, x_ext,
           ei_ind_txn_ext, ei_org_txn_ext, ei_ind_rev_txn_ext,
           ei_org_rev_txn_ext,
           ea_ind_txn_ext, ea_org_txn_ext, ea_ind_rev_txn_ext,
           ea_org_rev_txn_ext,
           W_nn_1, b_nn_1, W_root_1, bias_1,
           W_nn_2, b_nn_2, W_root_2, bias_2,
           W_nn_3, b_nn_3, W_root_3, bias_3,
           W_nn_4, b_nn_4, W_root_4, bias_4):
    wnns = [W_nn_1, W_nn_2, W_nn_3, W_nn_4]
    bnns = [b_nn_1, b_nn_2, b_nn_3, b_nn_4]

    wcats = []
    for wnn, bnn in zip(wnns, bnns):
        wt = wnn.reshape(D_EDGE, D_FEAT, OUT).transpose(1, 0, 2)
        wt = wt.reshape(D_FEAT, D_EDGE * OUT)
        bt = bnn.reshape(D_FEAT, OUT)
        wcats.append(jnp.concatenate(
            [wt, bt, jnp.zeros((D_FEAT, OUT), jnp.float32)], axis=1))
    wcat = jnp.stack(wcats)
    xs = jnp.stack([x_ind, x_org])
    wroot = jnp.concatenate([W_root_1, W_root_2, W_root_3, W_root_4], axis=1)
    bias = jnp.concatenate([bias_1, bias_2, bias_3, bias_4])
    biasp = jnp.broadcast_to(bias[None, :], (8, 4 * OUT))

    table = _build_tables(xs, wcat)
    part = _sc_edges(
        table.reshape(4 * N, ROW),
        [ei_ind_txn_ext.reshape(2 * E), ei_org_txn_ext.reshape(2 * E),
         ei_ind_rev_txn_ext.reshape(2 * E), ei_org_rev_txn_ext.reshape(2 * E)],
        [ea_ind_txn_ext.reshape(E * D_EDGE), ea_org_txn_ext.reshape(E * D_EDGE),
         ea_ind_rev_txn_ext.reshape(E * D_EDGE),
         ea_org_rev_txn_ext.reshape(E * D_EDGE)])
    return _combine(x_ext, wroot, biasp, part)
